# Initial kernel scaffold; baseline (speedup 1.0000x reference)
import jax
import jax.numpy as jnp
from jax.experimental import pallas as pl


def kernel(stacks, mask, W, b):
    raise NotImplementedError("write your pallas kernel here")


# speedup vs baseline: 1.0114x; 1.0114x over previous
import jax
import jax.numpy as jnp
from jax.experimental import pallas as pl
from jax.experimental.pallas import tpu as pltpu

_BC = 65536


def _tc_body(x_ref, k_ref, w_ref, b_ref, o_ref):
    x = x_ref[0]
    y = jax.lax.dot_general(
        w_ref[...], x, (((1,), (0,)), ((), ())),
        preferred_element_type=jnp.float32,
    )
    y = y + b_ref[...]
    o_ref[0] = y * k_ref[0]


def kernel(stacks, mask, W, b):
    bsz, num_stacks, n, n1 = stacks.shape
    nh = W.shape[0]
    S = n * n1
    sv = stacks.reshape(bsz, num_stacks, S)
    keep = 1.0 - mask.reshape(bsz, 1, S).astype(jnp.float32)
    b2 = b.reshape(nh, 1)
    grid = (bsz, S // _BC)
    out = pl.pallas_call(
        _tc_body,
        grid=grid,
        in_specs=[
            pl.BlockSpec((1, num_stacks, _BC), lambda bi, ci: (bi, 0, ci)),
            pl.BlockSpec((1, 1, _BC), lambda bi, ci: (bi, 0, ci)),
            pl.BlockSpec((nh, num_stacks), lambda bi, ci: (0, 0)),
            pl.BlockSpec((nh, 1), lambda bi, ci: (0, 0)),
        ],
        out_specs=pl.BlockSpec((1, nh, _BC), lambda bi, ci: (bi, 0, ci)),
        out_shape=jax.ShapeDtypeStruct((bsz, nh, S), jnp.float32),
        compiler_params=pltpu.CompilerParams(
            dimension_semantics=("parallel", "parallel"),
        ),
    )(sv, keep, W, b2)
    return out.reshape(bsz, nh, n, n1)


# speedup vs baseline: 1.6841x; 1.6651x over previous
import jax
import jax.numpy as jnp
from jax.experimental import pallas as pl
from jax.experimental.pallas import tpu as pltpu

_BR = 8


def _tc_body(x_ref, k_ref, w_ref, b_ref, o_ref):
    w = w_ref[...]
    bias = b_ref[...]
    for r in range(_BR):
        x = x_ref[0, :, r, :]
        y = jax.lax.dot_general(
            w, x, (((1,), (0,)), ((), ())),
            preferred_element_type=jnp.float32,
        )
        o_ref[0, :, r, :] = (y + bias) * k_ref[0, r][None, :]


def kernel(stacks, mask, W, b):
    bsz, num_stacks, n, n1 = stacks.shape
    nh = W.shape[0]
    keep = 1.0 - mask.astype(jnp.float32)
    b2 = b.reshape(nh, 1)
    grid = (bsz, n // _BR)
    out = pl.pallas_call(
        _tc_body,
        grid=grid,
        in_specs=[
            pl.BlockSpec((1, num_stacks, _BR, n1), lambda bi, ri: (bi, 0, ri, 0)),
            pl.BlockSpec((1, _BR, n1), lambda bi, ri: (bi, ri, 0)),
            pl.BlockSpec((nh, num_stacks), lambda bi, ri: (0, 0)),
            pl.BlockSpec((nh, 1), lambda bi, ri: (0, 0)),
        ],
        out_specs=pl.BlockSpec((1, nh, _BR, n1), lambda bi, ri: (bi, 0, ri, 0)),
        out_shape=jax.ShapeDtypeStruct((bsz, nh, n, n1), jnp.float32),
        compiler_params=pltpu.CompilerParams(
            dimension_semantics=("parallel", "parallel"),
        ),
    )(stacks, keep, W, b2)
    return out
